# baseline (speedup 1.0000x reference)
import functools

import jax
import jax.numpy as jnp
import numpy as np
from jax import lax
from jax.experimental import pallas as pl
from jax.experimental.pallas import tpu as pltpu
from jax.experimental.pallas import tpu_sc as plsc

N_NODES = 10000
N_EDGES = 320000
F = 128
EA_F = 16

NC, NS = 2, 16
NW = NC * NS
EPW = N_EDGES // NW
CH = 40
NCHUNK = EPW // CH
N_PAD = 10240
ZROWS = 32
ZREP = N_PAD // NS // ZROWS


_PERM = np.empty(F, dtype=np.int32)
for _g in range(F // 32):
    for _t in range(16):
        _PERM[32 * _g + 2 * _t] = 32 * _g + _t
        _PERM[32 * _g + 2 * _t + 1] = 32 * _g + 16 + _t


def _pq_body(nf_ref, wa_ref, wb_ref, p_ref, q_ref):
    x = nf_ref[...]
    p_ref[...] = jnp.dot(x, wa_ref[...], preferred_element_type=jnp.float32)
    q_ref[...] = jnp.dot(x, wb_ref[...], preferred_element_type=jnp.float32)


def _make_pq(nf, w1a, w1b):
    blk = 2000
    grid = N_NODES // blk
    return pl.pallas_call(
        _pq_body,
        grid=(grid,),
        in_specs=[
            pl.BlockSpec((blk, F), lambda i: (i, 0)),
            pl.BlockSpec((F, F), lambda i: (0, 0)),
            pl.BlockSpec((F, F), lambda i: (0, 0)),
        ],
        out_specs=[
            pl.BlockSpec((blk, F), lambda i: (i, 0)),
            pl.BlockSpec((blk, F), lambda i: (i, 0)),
        ],
        out_shape=[
            jax.ShapeDtypeStruct((N_NODES, F), jnp.float32),
            jax.ShapeDtypeStruct((N_NODES, F), jnp.float32),
        ],
    )(nf, w1a, w1b)


PACK = 128 // EA_F
EP_ROWS = N_EDGES // PACK
EP_OUT = PACK * F


def _e_body(ea_ref, wbd_ref, bt_ref, e_ref):
    blk = ea_ref.shape[0]
    packed = (
        jnp.dot(
            ea_ref[...].astype(jnp.bfloat16),
            wbd_ref[...],
            preferred_element_type=jnp.float32,
        )
        + bt_ref[...]
    )
    e_ref[...] = packed.reshape(blk * PACK, F)


def _make_e(ea_p, wbd, bt):
    blk = 2000
    grid = EP_ROWS // blk
    return pl.pallas_call(
        _e_body,
        grid=(grid,),
        in_specs=[
            pl.BlockSpec((blk, F), lambda i: (i, 0)),
            pl.BlockSpec((F, EP_OUT), lambda i: (0, 0)),
            pl.BlockSpec((1, EP_OUT), lambda i: (0, 0)),
        ],
        out_specs=pl.BlockSpec((blk * PACK, F), lambda i: (i, 0)),
        out_shape=jax.ShapeDtypeStruct((N_EDGES, F), jnp.float32),
    )(ea_p, wbd, bt)


def _sc_edge_body(row_hbm, col_hbm, p_hbm, q_hbm, e_hbm, ef_out, agg_out,
                  idxr0, idxc0, idxr1, idxc1, sidx0, sidx1,
                  acc_a0, acc_b0, acc_c0, acc_a1, acc_b1, acc_c1,
                  ob0, ob1, zbuf, agg_sh,
                  sem_i0, sem_i1, sem_g0, sem_g1, sem_w0, sem_w1,
                  sem_s0, sem_s1):
    cid = lax.axis_index("c")
    sid = lax.axis_index("s")
    wid = sid * NC + cid
    ebase = wid * EPW

    idxr = (idxr0, idxr1)
    idxc = (idxc0, idxc1)
    sidx = (sidx0, sidx1)
    acc_a = (acc_a0, acc_a1)
    acc_b = (acc_b0, acc_b1)
    acc_c = (acc_c0, acc_c1)
    ob = (ob0, ob1)
    sem_i = (sem_i0, sem_i1)
    sem_g = (sem_g0, sem_g1)
    sem_w = (sem_w0, sem_w1)
    sem_s = (sem_s0, sem_s1)

    zero16 = jnp.zeros((16,), jnp.float32)

    def _zfill(r, carry):
        for j in range(F // 16):
            zbuf[r, pl.ds(j * 16, 16)] = zero16
        return carry

    lax.fori_loop(0, ZROWS, _zfill, 0)
    for k in range(ZREP):
        pltpu.sync_copy(zbuf, agg_sh.at[pl.ds(sid * (ZROWS * ZREP) + k * ZROWS, ZROWS)])

    def issue_idx(c, b):
        sl = pl.ds(ebase + c * CH, CH)
        pltpu.async_copy(row_hbm.at[sl], idxr[b], sem_i[b])
        pltpu.async_copy(col_hbm.at[sl], idxc[b], sem_i[b])

    def wait_idx(c, b):
        sl = pl.ds(ebase + c * CH, CH)
        pltpu.make_async_copy(row_hbm.at[sl], idxr[b], sem_i[b]).wait()
        pltpu.make_async_copy(col_hbm.at[sl], idxc[b], sem_i[b]).wait()

    def issue_g(c, b):
        pltpu.async_copy(p_hbm.at[idxr[b]], acc_a[b], sem_g[b])
        pltpu.async_copy(q_hbm.at[idxc[b]], acc_b[b], sem_g[b])
        pltpu.async_copy(e_hbm.at[pl.ds(ebase + c * CH, CH)], acc_c[b], sem_g[b])

    def wait_g(c, b):
        pltpu.make_async_copy(p_hbm.at[idxr[b]], acc_a[b], sem_g[b]).wait()
        pltpu.make_async_copy(q_hbm.at[idxc[b]], acc_b[b], sem_g[b]).wait()
        pltpu.make_async_copy(e_hbm.at[pl.ds(ebase + c * CH, CH)], acc_c[b], sem_g[b]).wait()

    def copy_sidx(b):
        s, d = idxr[b], sidx[b]
        d[pl.ds(0, 16)] = s[pl.ds(0, 16)]
        d[pl.ds(16, 16)] = s[pl.ds(16, 16)]
        d[pl.ds(24, 16)] = s[pl.ds(24, 16)]

    def combine(b):
        a, bb, cc, o = acc_a[b], acc_b[b], acc_c[b], ob[b]

        @plsc.parallel_loop(0, CH, unroll=4)
        def _rl(r):
            for j in range(F // 16):
                sl = pl.ds(j * 16, 16)
                o[r, sl] = jnp.maximum(a[r, sl] + bb[r, sl] + cc[r, sl], 0.0)

    def write(c, b):
        pltpu.async_copy(ob[b], ef_out.at[pl.ds(ebase + c * CH, CH)], sem_w[b])
        pltpu.async_copy(ob[b], agg_sh.at[sidx[b]], sem_s[b], add=True)

    def wait_w(c, b):
        pltpu.make_async_copy(ob[b], ef_out.at[pl.ds(ebase + c * CH, CH)], sem_w[b]).wait()
        pltpu.make_async_copy(ob[b], agg_sh.at[sidx[b]], sem_s[b]).wait()

    plsc.subcore_barrier()

    issue_idx(0, 0)
    issue_idx(1, 1)
    wait_idx(0, 0)
    issue_g(0, 0)
    wait_idx(1, 1)
    issue_g(1, 1)
    wait_g(0, 0)
    copy_sidx(0)
    issue_idx(2, 0)
    combine(0)
    write(0, 0)
    wait_idx(2, 0)
    issue_g(2, 0)
    wait_g(1, 1)
    copy_sidx(1)
    issue_idx(3, 1)
    combine(1)
    write(1, 1)
    wait_idx(3, 1)
    issue_g(3, 1)

    def _round(k, carry):
        c0 = 2 * k
        wait_g(c0, 0)
        wait_w(c0 - 2, 0)
        copy_sidx(0)
        issue_idx(c0 + 2, 0)
        combine(0)
        write(c0, 0)
        wait_idx(c0 + 2, 0)
        issue_g(c0 + 2, 0)
        wait_g(c0 + 1, 1)
        wait_w(c0 - 1, 1)
        copy_sidx(1)
        issue_idx(c0 + 3, 1)
        combine(1)
        write(c0 + 1, 1)
        wait_idx(c0 + 3, 1)
        issue_g(c0 + 3, 1)
        return carry

    lax.fori_loop(1, NCHUNK // 2 - 1, _round, 0)

    cl = NCHUNK - 2
    wait_g(cl, 0)
    wait_w(cl - 2, 0)
    copy_sidx(0)
    combine(0)
    write(cl, 0)
    wait_g(cl + 1, 1)
    wait_w(cl - 1, 1)
    copy_sidx(1)
    combine(1)
    write(cl + 1, 1)
    wait_w(cl, 0)
    wait_w(cl + 1, 1)
    plsc.subcore_barrier()

    for k in range(ZREP):
        sl = pl.ds(sid * (ZROWS * ZREP) + k * ZROWS, ZROWS)
        pltpu.sync_copy(agg_sh.at[sl], zbuf)
        pltpu.sync_copy(zbuf, agg_out.at[cid].at[sl])


def _make_sc_edge(row, col, p, q, e):
    mesh = plsc.VectorSubcoreMesh(
        core_axis_name="c", subcore_axis_name="s", num_cores=NC, num_subcores=NS
    )
    f = pl.kernel(
        _sc_edge_body,
        out_type=(
            jax.ShapeDtypeStruct((N_EDGES, F), jnp.float32),
            jax.ShapeDtypeStruct((NC, N_PAD, F), jnp.float32),
        ),
        mesh=mesh,
        scratch_types=[
            pltpu.VMEM((CH,), jnp.int32),
            pltpu.VMEM((CH,), jnp.int32),
            pltpu.VMEM((CH,), jnp.int32),
            pltpu.VMEM((CH,), jnp.int32),
            pltpu.VMEM((CH,), jnp.int32),
            pltpu.VMEM((CH,), jnp.int32),
            pltpu.VMEM((CH, F), jnp.float32),
            pltpu.VMEM((CH, F), jnp.float32),
            pltpu.VMEM((CH, F), jnp.float32),
            pltpu.VMEM((CH, F), jnp.float32),
            pltpu.VMEM((CH, F), jnp.float32),
            pltpu.VMEM((CH, F), jnp.float32),
            pltpu.VMEM((CH, F), jnp.float32),
            pltpu.VMEM((CH, F), jnp.float32),
            pltpu.VMEM((ZROWS, F), jnp.float32),
            pltpu.MemorySpace.VMEM_SHARED((N_PAD, F), jnp.float32),
            pltpu.SemaphoreType.DMA,
            pltpu.SemaphoreType.DMA,
            pltpu.SemaphoreType.DMA,
            pltpu.SemaphoreType.DMA,
            pltpu.SemaphoreType.DMA,
            pltpu.SemaphoreType.DMA,
            pltpu.SemaphoreType.DMA,
            pltpu.SemaphoreType.DMA,
        ],
    )
    return f(row, col, p, q, e)


def _node_body(nf_ref, ap_ref, w2a_ref, w2b_ref, b2_ref, w3_ref, b3_ref, out_ref):
    x = nf_ref[...]
    agg = ap_ref[0] + ap_ref[1]
    h = jnp.maximum(
        jnp.dot(x, w2a_ref[...], preferred_element_type=jnp.float32)
        + jnp.dot(agg, w2b_ref[...], preferred_element_type=jnp.float32)
        + b2_ref[...],
        0.0,
    )
    out_ref[...] = (
        jnp.dot(h, w3_ref[...], preferred_element_type=jnp.float32) + b3_ref[...]
    )


def _make_node(nf, agg_pair, w2a, w2b, b2r, w3, b3r):
    blk = 2000
    grid = N_NODES // blk
    return pl.pallas_call(
        _node_body,
        grid=(grid,),
        in_specs=[
            pl.BlockSpec((blk, F), lambda i: (i, 0)),
            pl.BlockSpec((NC, blk, F), lambda i: (0, i, 0)),
            pl.BlockSpec((F, F), lambda i: (0, 0)),
            pl.BlockSpec((F, F), lambda i: (0, 0)),
            pl.BlockSpec((1, F), lambda i: (0, 0)),
            pl.BlockSpec((F, F), lambda i: (0, 0)),
            pl.BlockSpec((1, F), lambda i: (0, 0)),
        ],
        out_specs=pl.BlockSpec((blk, F), lambda i: (i, 0)),
        out_shape=jax.ShapeDtypeStruct((N_NODES, F), jnp.float32),
    )(nf, agg_pair, w2a, w2b, b2r, w3, b3r)


def kernel(node_feats, edge_index, edge_attr, W1, b1, W2, b2, W3, b3):
    ei = edge_index.astype(jnp.int32)
    row = ei[0]
    col = ei[1]

    w1a = W1[:F]
    w1b = W1[F : 2 * F]
    w1c = W1[2 * F :]
    wbd = jnp.kron(jnp.eye(PACK, dtype=jnp.float32), w1c).astype(jnp.bfloat16)
    bt = jnp.tile(b1, PACK).reshape(1, EP_OUT)
    ea_p = edge_attr.astype(jnp.bfloat16).reshape(EP_ROWS, F)
    w2a = W2[:F]
    w2b = W2[F:]
    b2r = b2.reshape(1, F)
    w3 = W3
    b3r = b3.reshape(1, F)

    p, q = _make_pq(node_feats, w1a, w1b)
    e = _make_e(ea_p, wbd, bt)
    edge_feats, agg_pair = _make_sc_edge(row, col, p, q, e)
    agg_pair = agg_pair[:, :N_NODES]
    node_out = _make_node(node_feats, agg_pair, w2a, w2b, b2r, w3, b3r)
    return (node_out, edge_feats)


# speedup vs baseline: 1.0124x; 1.0124x over previous
import functools

import jax
import jax.numpy as jnp
import numpy as np
from jax import lax
from jax.experimental import pallas as pl
from jax.experimental.pallas import tpu as pltpu
from jax.experimental.pallas import tpu_sc as plsc

N_NODES = 10000
N_EDGES = 320000
F = 128
EA_F = 16

NC, NS = 2, 16
NW = NC * NS
EPW = N_EDGES // NW
CH = 40
NCHUNK = EPW // CH
N_PAD = 10240
ZROWS = 32
ZREP = N_PAD // NS // ZROWS


_PERM = np.empty(F, dtype=np.int32)
for _g in range(F // 32):
    for _t in range(16):
        _PERM[32 * _g + 2 * _t] = 32 * _g + _t
        _PERM[32 * _g + 2 * _t + 1] = 32 * _g + 16 + _t


PACK = 128 // EA_F
EP_ROWS = N_EDGES // PACK
EP_OUT = PACK * F


_PQ_STEPS = 5


def _prep_body(ea_ref, wbd_ref, bt_ref, nf_ref, wa_ref, wb_ref,
               e_ref, p_ref, q_ref):
    i = pl.program_id(0)
    blk = ea_ref.shape[0]
    packed = (
        jnp.dot(
            ea_ref[...].astype(jnp.bfloat16),
            wbd_ref[...],
            preferred_element_type=jnp.float32,
        )
        + bt_ref[...]
    )
    e_ref[...] = packed.reshape(blk * PACK, F)

    @pl.when(i < _PQ_STEPS)
    def _():
        x = nf_ref[...]
        p_ref[...] = jnp.dot(x, wa_ref[...], preferred_element_type=jnp.float32)
        q_ref[...] = jnp.dot(x, wb_ref[...], preferred_element_type=jnp.float32)


def _make_prep(ea_p, wbd, bt, nf, w1a, w1b):
    blk = 2000
    grid = EP_ROWS // blk
    nblk = N_NODES // _PQ_STEPS
    clamp = lambda i: (jnp.minimum(i, _PQ_STEPS - 1), 0)
    return pl.pallas_call(
        _prep_body,
        grid=(grid,),
        in_specs=[
            pl.BlockSpec((blk, F), lambda i: (i, 0)),
            pl.BlockSpec((F, EP_OUT), lambda i: (0, 0)),
            pl.BlockSpec((1, EP_OUT), lambda i: (0, 0)),
            pl.BlockSpec((nblk, F), clamp),
            pl.BlockSpec((F, F), lambda i: (0, 0)),
            pl.BlockSpec((F, F), lambda i: (0, 0)),
        ],
        out_specs=[
            pl.BlockSpec((blk * PACK, F), lambda i: (i, 0)),
            pl.BlockSpec((nblk, F), clamp),
            pl.BlockSpec((nblk, F), clamp),
        ],
        out_shape=[
            jax.ShapeDtypeStruct((N_EDGES, F), jnp.float32),
            jax.ShapeDtypeStruct((N_NODES, F), jnp.float32),
            jax.ShapeDtypeStruct((N_NODES, F), jnp.float32),
        ],
    )(ea_p, wbd, bt, nf, w1a, w1b)


def _sc_edge_body(row_hbm, col_hbm, p_hbm, q_hbm, e_hbm, ef_out, agg_out,
                  idxr0, idxc0, idxr1, idxc1, sidx0, sidx1,
                  acc_a0, acc_b0, acc_c0, acc_a1, acc_b1, acc_c1,
                  ob0, ob1, zbuf, agg_sh,
                  sem_i0, sem_i1, sem_g0, sem_g1, sem_w0, sem_w1,
                  sem_s0, sem_s1):
    cid = lax.axis_index("c")
    sid = lax.axis_index("s")
    wid = sid * NC + cid
    ebase = wid * EPW

    idxr = (idxr0, idxr1)
    idxc = (idxc0, idxc1)
    sidx = (sidx0, sidx1)
    acc_a = (acc_a0, acc_a1)
    acc_b = (acc_b0, acc_b1)
    acc_c = (acc_c0, acc_c1)
    ob = (ob0, ob1)
    sem_i = (sem_i0, sem_i1)
    sem_g = (sem_g0, sem_g1)
    sem_w = (sem_w0, sem_w1)
    sem_s = (sem_s0, sem_s1)

    zero16 = jnp.zeros((16,), jnp.float32)

    def _zfill(r, carry):
        for j in range(F // 16):
            zbuf[r, pl.ds(j * 16, 16)] = zero16
        return carry

    lax.fori_loop(0, ZROWS, _zfill, 0)
    for k in range(ZREP):
        pltpu.sync_copy(zbuf, agg_sh.at[pl.ds(sid * (ZROWS * ZREP) + k * ZROWS, ZROWS)])

    def issue_idx(c, b):
        sl = pl.ds(ebase + c * CH, CH)
        pltpu.async_copy(row_hbm.at[sl], idxr[b], sem_i[b])
        pltpu.async_copy(col_hbm.at[sl], idxc[b], sem_i[b])

    def wait_idx(c, b):
        sl = pl.ds(ebase + c * CH, CH)
        pltpu.make_async_copy(row_hbm.at[sl], idxr[b], sem_i[b]).wait()
        pltpu.make_async_copy(col_hbm.at[sl], idxc[b], sem_i[b]).wait()

    def issue_g(c, b):
        pltpu.async_copy(p_hbm.at[idxr[b]], acc_a[b], sem_g[b])
        pltpu.async_copy(q_hbm.at[idxc[b]], acc_b[b], sem_g[b])
        pltpu.async_copy(e_hbm.at[pl.ds(ebase + c * CH, CH)], acc_c[b], sem_g[b])

    def wait_g(c, b):
        pltpu.make_async_copy(p_hbm.at[idxr[b]], acc_a[b], sem_g[b]).wait()
        pltpu.make_async_copy(q_hbm.at[idxc[b]], acc_b[b], sem_g[b]).wait()
        pltpu.make_async_copy(e_hbm.at[pl.ds(ebase + c * CH, CH)], acc_c[b], sem_g[b]).wait()

    def copy_sidx(b):
        s, d = idxr[b], sidx[b]
        d[pl.ds(0, 16)] = s[pl.ds(0, 16)]
        d[pl.ds(16, 16)] = s[pl.ds(16, 16)]
        d[pl.ds(24, 16)] = s[pl.ds(24, 16)]

    def combine(b):
        a, bb, cc, o = acc_a[b], acc_b[b], acc_c[b], ob[b]

        def _rl(r, carry):
            for j in range(F // 16):
                sl = pl.ds(j * 16, 16)
                o[r, sl] = jnp.maximum(a[r, sl] + bb[r, sl] + cc[r, sl], 0.0)
            return carry

        lax.fori_loop(0, CH, _rl, 0)

    def write(c, b):
        pltpu.async_copy(ob[b], ef_out.at[pl.ds(ebase + c * CH, CH)], sem_w[b])
        pltpu.async_copy(ob[b], agg_sh.at[sidx[b]], sem_s[b], add=True)

    def wait_w(c, b):
        pltpu.make_async_copy(ob[b], ef_out.at[pl.ds(ebase + c * CH, CH)], sem_w[b]).wait()
        pltpu.make_async_copy(ob[b], agg_sh.at[sidx[b]], sem_s[b]).wait()

    plsc.subcore_barrier()

    issue_idx(0, 0)
    issue_idx(1, 1)
    wait_idx(0, 0)
    issue_g(0, 0)
    wait_idx(1, 1)
    issue_g(1, 1)
    wait_g(0, 0)
    copy_sidx(0)
    issue_idx(2, 0)
    combine(0)
    write(0, 0)
    wait_idx(2, 0)
    issue_g(2, 0)
    wait_g(1, 1)
    copy_sidx(1)
    issue_idx(3, 1)
    combine(1)
    write(1, 1)
    wait_idx(3, 1)
    issue_g(3, 1)

    def _round(k, carry):
        c0 = 2 * k
        wait_g(c0, 0)
        wait_w(c0 - 2, 0)
        copy_sidx(0)
        issue_idx(c0 + 2, 0)
        combine(0)
        write(c0, 0)
        wait_idx(c0 + 2, 0)
        issue_g(c0 + 2, 0)
        wait_g(c0 + 1, 1)
        wait_w(c0 - 1, 1)
        copy_sidx(1)
        issue_idx(c0 + 3, 1)
        combine(1)
        write(c0 + 1, 1)
        wait_idx(c0 + 3, 1)
        issue_g(c0 + 3, 1)
        return carry

    lax.fori_loop(1, NCHUNK // 2 - 1, _round, 0)

    cl = NCHUNK - 2
    wait_g(cl, 0)
    wait_w(cl - 2, 0)
    copy_sidx(0)
    combine(0)
    write(cl, 0)
    wait_g(cl + 1, 1)
    wait_w(cl - 1, 1)
    copy_sidx(1)
    combine(1)
    write(cl + 1, 1)
    wait_w(cl, 0)
    wait_w(cl + 1, 1)
    plsc.subcore_barrier()

    for k in range(ZREP):
        sl = pl.ds(sid * (ZROWS * ZREP) + k * ZROWS, ZROWS)
        pltpu.sync_copy(agg_sh.at[sl], zbuf)
        pltpu.sync_copy(zbuf, agg_out.at[cid].at[sl])


def _make_sc_edge(row, col, p, q, e):
    mesh = plsc.VectorSubcoreMesh(
        core_axis_name="c", subcore_axis_name="s", num_cores=NC, num_subcores=NS
    )
    f = pl.kernel(
        _sc_edge_body,
        out_type=(
            jax.ShapeDtypeStruct((N_EDGES, F), jnp.float32),
            jax.ShapeDtypeStruct((NC, N_PAD, F), jnp.float32),
        ),
        mesh=mesh,
        scratch_types=[
            pltpu.VMEM((CH,), jnp.int32),
            pltpu.VMEM((CH,), jnp.int32),
            pltpu.VMEM((CH,), jnp.int32),
            pltpu.VMEM((CH,), jnp.int32),
            pltpu.VMEM((CH,), jnp.int32),
            pltpu.VMEM((CH,), jnp.int32),
            pltpu.VMEM((CH, F), jnp.float32),
            pltpu.VMEM((CH, F), jnp.float32),
            pltpu.VMEM((CH, F), jnp.float32),
            pltpu.VMEM((CH, F), jnp.float32),
            pltpu.VMEM((CH, F), jnp.float32),
            pltpu.VMEM((CH, F), jnp.float32),
            pltpu.VMEM((CH, F), jnp.float32),
            pltpu.VMEM((CH, F), jnp.float32),
            pltpu.VMEM((ZROWS, F), jnp.float32),
            pltpu.MemorySpace.VMEM_SHARED((N_PAD, F), jnp.float32),
            pltpu.SemaphoreType.DMA,
            pltpu.SemaphoreType.DMA,
            pltpu.SemaphoreType.DMA,
            pltpu.SemaphoreType.DMA,
            pltpu.SemaphoreType.DMA,
            pltpu.SemaphoreType.DMA,
            pltpu.SemaphoreType.DMA,
            pltpu.SemaphoreType.DMA,
        ],
    )
    return f(row, col, p, q, e)


def _node_body(nf_ref, ap_ref, w2a_ref, w2b_ref, b2_ref, w3_ref, b3_ref, out_ref):
    x = nf_ref[...]
    agg = ap_ref[0] + ap_ref[1]
    h = jnp.maximum(
        jnp.dot(x, w2a_ref[...], preferred_element_type=jnp.float32)
        + jnp.dot(agg, w2b_ref[...], preferred_element_type=jnp.float32)
        + b2_ref[...],
        0.0,
    )
    out_ref[...] = (
        jnp.dot(h, w3_ref[...], preferred_element_type=jnp.float32) + b3_ref[...]
    )


def _make_node(nf, agg_pair, w2a, w2b, b2r, w3, b3r):
    blk = 2000
    grid = N_NODES // blk
    return pl.pallas_call(
        _node_body,
        grid=(grid,),
        in_specs=[
            pl.BlockSpec((blk, F), lambda i: (i, 0)),
            pl.BlockSpec((NC, blk, F), lambda i: (0, i, 0)),
            pl.BlockSpec((F, F), lambda i: (0, 0)),
            pl.BlockSpec((F, F), lambda i: (0, 0)),
            pl.BlockSpec((1, F), lambda i: (0, 0)),
            pl.BlockSpec((F, F), lambda i: (0, 0)),
            pl.BlockSpec((1, F), lambda i: (0, 0)),
        ],
        out_specs=pl.BlockSpec((blk, F), lambda i: (i, 0)),
        out_shape=jax.ShapeDtypeStruct((N_NODES, F), jnp.float32),
    )(nf, agg_pair, w2a, w2b, b2r, w3, b3r)


def kernel(node_feats, edge_index, edge_attr, W1, b1, W2, b2, W3, b3):
    ei = edge_index.astype(jnp.int32)
    row = ei[0]
    col = ei[1]

    w1a = W1[:F]
    w1b = W1[F : 2 * F]
    w1c = W1[2 * F :]
    wbd = jnp.kron(jnp.eye(PACK, dtype=jnp.float32), w1c).astype(jnp.bfloat16)
    bt = jnp.tile(b1, PACK).reshape(1, EP_OUT)
    ea_p = edge_attr.astype(jnp.bfloat16).reshape(EP_ROWS, F)
    w2a = W2[:F]
    w2b = W2[F:]
    b2r = b2.reshape(1, F)
    w3 = W3
    b3r = b3.reshape(1, F)

    e, p, q = _make_prep(ea_p, wbd, bt, node_feats, w1a, w1b)
    edge_feats, agg_pair = _make_sc_edge(row, col, p, q, e)
    agg_pair = agg_pair[:, :N_NODES]
    node_out = _make_node(node_feats, agg_pair, w2a, w2b, b2r, w3, b3r)
    return (node_out, edge_feats)


# speedup vs baseline: 1.0169x; 1.0045x over previous
import functools

import jax
import jax.numpy as jnp
import numpy as np
from jax import lax
from jax.experimental import pallas as pl
from jax.experimental.pallas import tpu as pltpu
from jax.experimental.pallas import tpu_sc as plsc

N_NODES = 10000
N_EDGES = 320000
F = 128
EA_F = 16

NC, NS = 2, 16
NW = NC * NS
EPW = N_EDGES // NW
CH = 40
NCHUNK = EPW // CH
N_PAD = 10240
ZROWS = 32
ZREP = N_PAD // NS // ZROWS


_PERM = np.empty(F, dtype=np.int32)
for _g in range(F // 32):
    for _t in range(16):
        _PERM[32 * _g + 2 * _t] = 32 * _g + _t
        _PERM[32 * _g + 2 * _t + 1] = 32 * _g + 16 + _t


PACK = 128 // EA_F
EP_ROWS = N_EDGES // PACK
EP_OUT = PACK * F


_PQ_STEPS = 5


def _prep_body(ea_ref, wbd_ref, bt_ref, nf_ref, wa_ref, wb_ref,
               e_ref, p_ref, q_ref):
    i = pl.program_id(0)
    blk = ea_ref.shape[0]
    packed = (
        jnp.dot(
            ea_ref[...].astype(jnp.bfloat16),
            wbd_ref[...],
            preferred_element_type=jnp.float32,
        )
        + bt_ref[...]
    )
    e_ref[...] = packed.reshape(blk * PACK, F)

    @pl.when(i < _PQ_STEPS)
    def _():
        x = nf_ref[...]
        p_ref[...] = jnp.dot(x, wa_ref[...], preferred_element_type=jnp.float32)
        q_ref[...] = jnp.dot(x, wb_ref[...], preferred_element_type=jnp.float32)


def _make_prep(ea_p, wbd, bt, nf, w1a, w1b):
    blk = 4000
    grid = EP_ROWS // blk
    nblk = N_NODES // _PQ_STEPS
    clamp = lambda i: (jnp.minimum(i, _PQ_STEPS - 1), 0)
    return pl.pallas_call(
        _prep_body,
        grid=(grid,),
        in_specs=[
            pl.BlockSpec((blk, F), lambda i: (i, 0)),
            pl.BlockSpec((F, EP_OUT), lambda i: (0, 0)),
            pl.BlockSpec((1, EP_OUT), lambda i: (0, 0)),
            pl.BlockSpec((nblk, F), clamp),
            pl.BlockSpec((F, F), lambda i: (0, 0)),
            pl.BlockSpec((F, F), lambda i: (0, 0)),
        ],
        out_specs=[
            pl.BlockSpec((blk * PACK, F), lambda i: (i, 0)),
            pl.BlockSpec((nblk, F), clamp),
            pl.BlockSpec((nblk, F), clamp),
        ],
        out_shape=[
            jax.ShapeDtypeStruct((N_EDGES, F), jnp.float32),
            jax.ShapeDtypeStruct((N_NODES, F), jnp.float32),
            jax.ShapeDtypeStruct((N_NODES, F), jnp.float32),
        ],
    )(ea_p, wbd, bt, nf, w1a, w1b)


def _sc_edge_body(row_hbm, col_hbm, p_hbm, q_hbm, e_hbm, ef_out, agg_out,
                  idxr0, idxc0, idxr1, idxc1, sidx0, sidx1,
                  acc_a0, acc_b0, acc_c0, acc_a1, acc_b1, acc_c1,
                  ob0, ob1, zbuf, agg_sh,
                  sem_i0, sem_i1, sem_g0, sem_g1, sem_w0, sem_w1,
                  sem_s0, sem_s1):
    cid = lax.axis_index("c")
    sid = lax.axis_index("s")
    wid = sid * NC + cid
    ebase = wid * EPW

    idxr = (idxr0, idxr1)
    idxc = (idxc0, idxc1)
    sidx = (sidx0, sidx1)
    acc_a = (acc_a0, acc_a1)
    acc_b = (acc_b0, acc_b1)
    acc_c = (acc_c0, acc_c1)
    ob = (ob0, ob1)
    sem_i = (sem_i0, sem_i1)
    sem_g = (sem_g0, sem_g1)
    sem_w = (sem_w0, sem_w1)
    sem_s = (sem_s0, sem_s1)

    zero16 = jnp.zeros((16,), jnp.float32)

    def _zfill(r, carry):
        for j in range(F // 16):
            zbuf[r, pl.ds(j * 16, 16)] = zero16
        return carry

    lax.fori_loop(0, ZROWS, _zfill, 0)
    for k in range(ZREP):
        pltpu.sync_copy(zbuf, agg_sh.at[pl.ds(sid * (ZROWS * ZREP) + k * ZROWS, ZROWS)])

    def issue_idx(c, b):
        sl = pl.ds(ebase + c * CH, CH)
        pltpu.async_copy(row_hbm.at[sl], idxr[b], sem_i[b])
        pltpu.async_copy(col_hbm.at[sl], idxc[b], sem_i[b])

    def wait_idx(c, b):
        sl = pl.ds(ebase + c * CH, CH)
        pltpu.make_async_copy(row_hbm.at[sl], idxr[b], sem_i[b]).wait()
        pltpu.make_async_copy(col_hbm.at[sl], idxc[b], sem_i[b]).wait()

    def issue_g(c, b):
        pltpu.async_copy(p_hbm.at[idxr[b]], acc_a[b], sem_g[b])
        pltpu.async_copy(q_hbm.at[idxc[b]], acc_b[b], sem_g[b])
        pltpu.async_copy(e_hbm.at[pl.ds(ebase + c * CH, CH)], acc_c[b], sem_g[b])

    def wait_g(c, b):
        pltpu.make_async_copy(p_hbm.at[idxr[b]], acc_a[b], sem_g[b]).wait()
        pltpu.make_async_copy(q_hbm.at[idxc[b]], acc_b[b], sem_g[b]).wait()
        pltpu.make_async_copy(e_hbm.at[pl.ds(ebase + c * CH, CH)], acc_c[b], sem_g[b]).wait()

    def copy_sidx(b):
        s, d = idxr[b], sidx[b]
        d[pl.ds(0, 16)] = s[pl.ds(0, 16)]
        d[pl.ds(16, 16)] = s[pl.ds(16, 16)]
        d[pl.ds(24, 16)] = s[pl.ds(24, 16)]

    def combine(b):
        a, bb, cc, o = acc_a[b], acc_b[b], acc_c[b], ob[b]

        def _rl(r, carry):
            for j in range(F // 16):
                sl = pl.ds(j * 16, 16)
                o[r, sl] = jnp.maximum(a[r, sl] + bb[r, sl] + cc[r, sl], 0.0)
            return carry

        lax.fori_loop(0, CH, _rl, 0)

    def write(c, b):
        pltpu.async_copy(ob[b], ef_out.at[pl.ds(ebase + c * CH, CH)], sem_w[b])
        pltpu.async_copy(ob[b], agg_sh.at[sidx[b]], sem_s[b], add=True)

    def wait_w(c, b):
        pltpu.make_async_copy(ob[b], ef_out.at[pl.ds(ebase + c * CH, CH)], sem_w[b]).wait()
        pltpu.make_async_copy(ob[b], agg_sh.at[sidx[b]], sem_s[b]).wait()

    plsc.subcore_barrier()

    issue_idx(0, 0)
    issue_idx(1, 1)
    wait_idx(0, 0)
    issue_g(0, 0)
    wait_idx(1, 1)
    issue_g(1, 1)
    wait_g(0, 0)
    copy_sidx(0)
    issue_idx(2, 0)
    combine(0)
    write(0, 0)
    wait_idx(2, 0)
    issue_g(2, 0)
    wait_g(1, 1)
    copy_sidx(1)
    issue_idx(3, 1)
    combine(1)
    write(1, 1)
    wait_idx(3, 1)
    issue_g(3, 1)

    def _round(k, carry):
        c0 = 2 * k
        wait_g(c0, 0)
        wait_w(c0 - 2, 0)
        copy_sidx(0)
        issue_idx(c0 + 2, 0)
        combine(0)
        write(c0, 0)
        wait_idx(c0 + 2, 0)
        issue_g(c0 + 2, 0)
        wait_g(c0 + 1, 1)
        wait_w(c0 - 1, 1)
        copy_sidx(1)
        issue_idx(c0 + 3, 1)
        combine(1)
        write(c0 + 1, 1)
        wait_idx(c0 + 3, 1)
        issue_g(c0 + 3, 1)
        return carry

    lax.fori_loop(1, NCHUNK // 2 - 1, _round, 0)

    cl = NCHUNK - 2
    wait_g(cl, 0)
    wait_w(cl - 2, 0)
    copy_sidx(0)
    combine(0)
    write(cl, 0)
    wait_g(cl + 1, 1)
    wait_w(cl - 1, 1)
    copy_sidx(1)
    combine(1)
    write(cl + 1, 1)
    wait_w(cl, 0)
    wait_w(cl + 1, 1)
    plsc.subcore_barrier()

    for k in range(ZREP):
        sl = pl.ds(sid * (ZROWS * ZREP) + k * ZROWS, ZROWS)
        pltpu.sync_copy(agg_sh.at[sl], zbuf)
        pltpu.sync_copy(zbuf, agg_out.at[cid].at[sl])


def _make_sc_edge(row, col, p, q, e):
    mesh = plsc.VectorSubcoreMesh(
        core_axis_name="c", subcore_axis_name="s", num_cores=NC, num_subcores=NS
    )
    f = pl.kernel(
        _sc_edge_body,
        out_type=(
            jax.ShapeDtypeStruct((N_EDGES, F), jnp.float32),
            jax.ShapeDtypeStruct((NC, N_PAD, F), jnp.float32),
        ),
        mesh=mesh,
        scratch_types=[
            pltpu.VMEM((CH,), jnp.int32),
            pltpu.VMEM((CH,), jnp.int32),
            pltpu.VMEM((CH,), jnp.int32),
            pltpu.VMEM((CH,), jnp.int32),
            pltpu.VMEM((CH,), jnp.int32),
            pltpu.VMEM((CH,), jnp.int32),
            pltpu.VMEM((CH, F), jnp.float32),
            pltpu.VMEM((CH, F), jnp.float32),
            pltpu.VMEM((CH, F), jnp.float32),
            pltpu.VMEM((CH, F), jnp.float32),
            pltpu.VMEM((CH, F), jnp.float32),
            pltpu.VMEM((CH, F), jnp.float32),
            pltpu.VMEM((CH, F), jnp.float32),
            pltpu.VMEM((CH, F), jnp.float32),
            pltpu.VMEM((ZROWS, F), jnp.float32),
            pltpu.MemorySpace.VMEM_SHARED((N_PAD, F), jnp.float32),
            pltpu.SemaphoreType.DMA,
            pltpu.SemaphoreType.DMA,
            pltpu.SemaphoreType.DMA,
            pltpu.SemaphoreType.DMA,
            pltpu.SemaphoreType.DMA,
            pltpu.SemaphoreType.DMA,
            pltpu.SemaphoreType.DMA,
            pltpu.SemaphoreType.DMA,
        ],
    )
    return f(row, col, p, q, e)


def _node_body(nf_ref, ap_ref, w2a_ref, w2b_ref, b2_ref, w3_ref, b3_ref, out_ref):
    x = nf_ref[...]
    agg = ap_ref[0] + ap_ref[1]
    h = jnp.maximum(
        jnp.dot(x, w2a_ref[...], preferred_element_type=jnp.float32)
        + jnp.dot(agg, w2b_ref[...], preferred_element_type=jnp.float32)
        + b2_ref[...],
        0.0,
    )
    out_ref[...] = (
        jnp.dot(h, w3_ref[...], preferred_element_type=jnp.float32) + b3_ref[...]
    )


def _make_node(nf, agg_pair, w2a, w2b, b2r, w3, b3r):
    blk = 2000
    grid = N_NODES // blk
    return pl.pallas_call(
        _node_body,
        grid=(grid,),
        in_specs=[
            pl.BlockSpec((blk, F), lambda i: (i, 0)),
            pl.BlockSpec((NC, blk, F), lambda i: (0, i, 0)),
            pl.BlockSpec((F, F), lambda i: (0, 0)),
            pl.BlockSpec((F, F), lambda i: (0, 0)),
            pl.BlockSpec((1, F), lambda i: (0, 0)),
            pl.BlockSpec((F, F), lambda i: (0, 0)),
            pl.BlockSpec((1, F), lambda i: (0, 0)),
        ],
        out_specs=pl.BlockSpec((blk, F), lambda i: (i, 0)),
        out_shape=jax.ShapeDtypeStruct((N_NODES, F), jnp.float32),
    )(nf, agg_pair, w2a, w2b, b2r, w3, b3r)


def kernel(node_feats, edge_index, edge_attr, W1, b1, W2, b2, W3, b3):
    ei = edge_index.astype(jnp.int32)
    row = ei[0]
    col = ei[1]

    w1a = W1[:F]
    w1b = W1[F : 2 * F]
    w1c = W1[2 * F :]
    wbd = jnp.kron(jnp.eye(PACK, dtype=jnp.float32), w1c).astype(jnp.bfloat16)
    bt = jnp.tile(b1, PACK).reshape(1, EP_OUT)
    ea_p = edge_attr.astype(jnp.bfloat16).reshape(EP_ROWS, F)
    w2a = W2[:F]
    w2b = W2[F:]
    b2r = b2.reshape(1, F)
    w3 = W3
    b3r = b3.reshape(1, F)

    e, p, q = _make_prep(ea_p, wbd, bt, node_feats, w1a, w1b)
    edge_feats, agg_pair = _make_sc_edge(row, col, p, q, e)
    agg_pair = agg_pair[:, :N_NODES]
    node_out = _make_node(node_feats, agg_pair, w2a, w2b, b2r, w3, b3r)
    return (node_out, edge_feats)
